# Initial kernel scaffold; baseline (speedup 1.0000x reference)
import jax
import jax.numpy as jnp
from jax.experimental import pallas as pl


def kernel(inputs, token_table, pos_table):
    raise NotImplementedError("write your pallas kernel here")


# speedup vs baseline: 3.6265x; 3.6265x over previous
import functools

import jax
import jax.numpy as jnp
from jax import lax
from jax.experimental import pallas as pl
from jax.experimental.pallas import tpu as pltpu
from jax.experimental.pallas import tpu_sc as plsc

VOCAB_SIZE = 100000
EMBED_DIM = 128
MAXLEN = 200
BATCH = 1024

NUM_CORES = 2
NUM_SUBCORES = 16
NUM_WORKERS = NUM_CORES * NUM_SUBCORES

SUBGATHER = 100
SUBS = MAXLEN // SUBGATHER
ROWS_PER_WORKER = BATCH // NUM_WORKERS
LANES = 16
VECS_PER_ROW = EMBED_DIM // LANES


def _emb_kernel(idx_hbm, token_hbm, pos_hbm, out_hbm, pos_v, idx_v, rows_v, sem):
    wid = lax.axis_index("s") * NUM_CORES + lax.axis_index("c")
    pltpu.sync_copy(pos_hbm, pos_v)

    def batch_body(j, carry):
        b = wid * ROWS_PER_WORKER + j
        pltpu.sync_copy(idx_hbm.at[b], idx_v)
        for h in range(SUBS):
            pltpu.async_copy(
                token_hbm.at[idx_v.at[h]],
                rows_v.at[pl.ds(h * SUBGATHER, SUBGATHER)],
                sem,
            ).wait()

        def row_body(r, c2):
            for v in range(VECS_PER_ROW):
                sl = pl.ds(v * LANES, LANES)
                plsc.addupdate(rows_v.at[r, sl], pos_v[r, sl])
            return c2

        lax.fori_loop(0, MAXLEN, row_body, 0, unroll=False)
        pltpu.sync_copy(rows_v, out_hbm.at[pl.ds(b * MAXLEN, MAXLEN)])
        return carry

    lax.fori_loop(0, ROWS_PER_WORKER, batch_body, 0, unroll=False)


@functools.partial(jax.jit, static_argnames=())
def kernel(inputs, token_table, pos_table):
    idx = inputs.reshape(BATCH, SUBS, SUBGATHER).astype(jnp.int32)
    mesh = plsc.VectorSubcoreMesh(core_axis_name="c", subcore_axis_name="s")
    out = pl.kernel(
        _emb_kernel,
        mesh=mesh,
        out_type=jax.ShapeDtypeStruct((BATCH * MAXLEN, EMBED_DIM), jnp.float32),
        scratch_types=[
            pltpu.VMEM((MAXLEN, EMBED_DIM), jnp.float32),
            pltpu.VMEM((SUBS, SUBGATHER), jnp.int32),
            pltpu.VMEM((MAXLEN, EMBED_DIM), jnp.float32),
            pltpu.SemaphoreType.DMA,
        ],
    )(idx, token_table, pos_table)
    return out.reshape(BATCH, MAXLEN, EMBED_DIM)


# speedup vs baseline: 6.3495x; 1.7509x over previous
import functools

import jax
import jax.numpy as jnp
from jax import lax
from jax.experimental import pallas as pl
from jax.experimental.pallas import tpu as pltpu
from jax.experimental.pallas import tpu_sc as plsc

VOCAB_SIZE = 100000
EMBED_DIM = 128
MAXLEN = 200
BATCH = 1024

NUM_CORES = 2
NUM_SUBCORES = 16
NUM_WORKERS = NUM_CORES * NUM_SUBCORES

SUBGATHER = 100
SUBS = MAXLEN // SUBGATHER
N = BATCH // NUM_WORKERS
LANES = 16
VECS_PER_ROW = EMBED_DIM // LANES


def _emb_kernel(idx_hbm, token_hbm, pos_hbm, out_hbm,
                pos_v, idx0, idx1, rows0, rows1,
                isem0, isem1, gsem0, gsem1, ssem0, ssem1):
    wid = lax.axis_index("s") * NUM_CORES + lax.axis_index("c")
    base = wid * N
    idx_b = (idx0, idx1)
    rows_b = (rows0, rows1)
    isem_b = (isem0, isem1)
    gsem_b = (gsem0, gsem1)
    ssem_b = (ssem0, ssem1)

    pltpu.sync_copy(pos_hbm, pos_v)

    def start_gathers(p, j):
        return [
            pltpu.async_copy(
                token_hbm.at[idx_b[p].at[h]],
                rows_b[p].at[pl.ds(h * SUBGATHER, SUBGATHER)],
                gsem_b[p],
            )
            for h in range(SUBS)
        ]

    def add_pos(p):
        def row_body(r, c2):
            for v in range(VECS_PER_ROW):
                sl = pl.ds(v * LANES, LANES)
                plsc.addupdate(rows_b[p].at[r, sl], pos_v[r, sl])
            return c2

        lax.fori_loop(0, MAXLEN, row_body, 0, unroll=False)

    pltpu.sync_copy(idx_hbm.at[base], idx0)
    g = {0: start_gathers(0, 0)}
    i = {1: pltpu.async_copy(idx_hbm.at[base + 1], idx1, isem1)}
    s = {}
    for j in range(N):
        p = j & 1
        q = p ^ 1
        if j + 1 < N:
            if j >= 1:
                s[j - 1].wait()
            i[j + 1].wait()
            g[j + 1] = start_gathers(q, j + 1)
        for d in g[j]:
            d.wait()
        if j + 2 < N:
            i[j + 2] = pltpu.async_copy(idx_hbm.at[base + j + 2], idx_b[p], isem_b[p])
        add_pos(p)
        s[j] = pltpu.async_copy(
            rows_b[p], out_hbm.at[pl.ds((base + j) * MAXLEN, MAXLEN)], ssem_b[p]
        )
    s[N - 2].wait()
    s[N - 1].wait()


@functools.partial(jax.jit, static_argnames=())
def kernel(inputs, token_table, pos_table):
    idx = inputs.reshape(BATCH, SUBS, SUBGATHER).astype(jnp.int32)
    mesh = plsc.VectorSubcoreMesh(core_axis_name="c", subcore_axis_name="s")
    out = pl.kernel(
        _emb_kernel,
        mesh=mesh,
        out_type=jax.ShapeDtypeStruct((BATCH * MAXLEN, EMBED_DIM), jnp.float32),
        scratch_types=[
            pltpu.VMEM((MAXLEN, EMBED_DIM), jnp.float32),
            pltpu.VMEM((SUBS, SUBGATHER), jnp.int32),
            pltpu.VMEM((SUBS, SUBGATHER), jnp.int32),
            pltpu.VMEM((MAXLEN, EMBED_DIM), jnp.float32),
            pltpu.VMEM((MAXLEN, EMBED_DIM), jnp.float32),
            pltpu.SemaphoreType.DMA,
            pltpu.SemaphoreType.DMA,
            pltpu.SemaphoreType.DMA,
            pltpu.SemaphoreType.DMA,
            pltpu.SemaphoreType.DMA,
            pltpu.SemaphoreType.DMA,
        ],
    )(idx, token_table, pos_table)
    return out.reshape(BATCH, MAXLEN, EMBED_DIM)
